# Initial kernel scaffold; baseline (speedup 1.0000x reference)
import jax
import jax.numpy as jnp
from jax.experimental import pallas as pl


def kernel(pred, scores, flag):
    raise NotImplementedError("write your pallas kernel here")


# speedup vs baseline: 1.4459x; 1.4459x over previous
import jax
import jax.numpy as jnp
from jax import lax
from jax.experimental import pallas as pl

BEAM = 16
BLANK_ID = 0
TEMPERATURE = 1.0
NEG_INF = float("-inf")


def _block_kernel(pred_ref, scores_ref, flag_ref, out_scores_ref, out_tokens_ref):
    x = pred_ref[...]
    beam, vocab = x.shape
    scores = scores_ref[...]
    flag = flag_ref[...]

    m = jnp.max(x, axis=1, keepdims=True)
    s = jnp.sum(jnp.exp(x - m), axis=1, keepdims=True)
    lse = m + jnp.log(s)

    col = lax.broadcasted_iota(jnp.int32, (beam, vocab), 1)
    xm = jnp.where(col == BLANK_ID, NEG_INF, x)

    vals = [x[:, BLANK_ID:BLANK_ID + 1]]
    idxs = [jnp.zeros((beam, 1), jnp.int32)]
    for _ in range(BEAM - 1):
        v = jnp.max(xm, axis=1, keepdims=True)
        a = jnp.min(jnp.where(xm == v, col, vocab), axis=1, keepdims=True)
        vals.append(v)
        idxs.append(a)
        xm = jnp.where(col == a, NEG_INF, xm)

    cand_vals = jnp.concatenate(vals, axis=1)
    cand_idx = jnp.concatenate(idxs, axis=1)
    cand = scores + cand_vals - lse
    cand = jnp.where(flag > 0, NEG_INF, cand)

    flat_id = lax.broadcasted_iota(jnp.int32, (beam, BEAM), 0) * BEAM + \
        lax.broadcasted_iota(jnp.int32, (beam, BEAM), 1)
    c = cand
    out_s = []
    out_t = []
    for _ in range(BEAM):
        v = jnp.max(c)
        sel = jnp.min(jnp.where(c == v, flat_id, BEAM * BEAM))
        tok = jnp.sum(jnp.where(flat_id == sel, cand_idx, 0))
        out_s.append(v)
        out_t.append(tok)
        c = jnp.where(flat_id == sel, NEG_INF, c)

    out_scores_ref[...] = jnp.stack(out_s).reshape(beam, 1)
    out_tokens_ref[...] = jnp.stack(out_t).astype(jnp.int32).reshape(beam, 1)


@jax.jit
def kernel(pred, scores, flag):
    rows, vocab = pred.shape
    batch = rows // BEAM
    flag_i = flag.astype(jnp.int32)

    grid = (batch,)
    out_scores, out_tokens = pl.pallas_call(
        _block_kernel,
        grid=grid,
        in_specs=[
            pl.BlockSpec((BEAM, vocab), lambda b: (b, 0)),
            pl.BlockSpec((BEAM, 1), lambda b: (b, 0)),
            pl.BlockSpec((BEAM, 1), lambda b: (b, 0)),
        ],
        out_specs=[
            pl.BlockSpec((BEAM, 1), lambda b: (b, 0)),
            pl.BlockSpec((BEAM, 1), lambda b: (b, 0)),
        ],
        out_shape=[
            jax.ShapeDtypeStruct((rows, 1), jnp.float32),
            jax.ShapeDtypeStruct((rows, 1), jnp.int32),
        ],
    )(pred / TEMPERATURE, scores, flag_i)
    return out_scores, out_tokens
